# baseline (speedup 1.0000x reference)
import functools

import jax
import jax.numpy as jnp
from jax.experimental import pallas as pl
from jax.experimental.pallas import tpu as pltpu

_B = 2
_N = 4096
_K = 32
_RADIUS = 0.15
_TI = 128
_CW = 256


def _ppf_tile(x_all_ref, xi_ref, ni_ref, out_ref):
    x_all = x_all_ref[0]
    xi = xi_ref[0]
    ni = ni_ref[0]

    xj0 = x_all[0:1, :]
    xj1 = x_all[1:2, :]
    xj2 = x_all[2:3, :]
    xi0 = xi[:, 0:1]
    xi1 = xi[:, 1:2]
    xi2 = xi[:, 2:3]
    n0 = ni[:, 0:1]
    n1 = ni[:, 1:2]
    n2 = ni[:, 2:3]

    sq_j = xj0 * xj0 + xj1 * xj1 + xj2 * xj2
    sq_i = xi0 * xi0 + xi1 * xi1 + xi2 * xi2
    dotx = jax.lax.dot_general(xi, x_all, (((1,), (0,)), ((), ())))
    d2 = jnp.maximum(sq_i + sq_j - 2.0 * dotx, 0.0)
    mask = d2 <= _RADIUS * _RADIUS

    ncw = _N // _CW
    tri = (jax.lax.broadcasted_iota(jnp.int32, (_CW, _CW), 0)
           <= jax.lax.broadcasted_iota(jnp.int32, (_CW, _CW), 1))
    tri = tri.astype(jnp.float32)
    locals_ = []
    totals = []
    for c in range(ncw):
        mc = mask[:, c * _CW:(c + 1) * _CW].astype(jnp.float32)
        loc = jax.lax.dot_general(mc, tri, (((1,), (0,)), ((), ())),
                                  preferred_element_type=jnp.float32)
        locals_.append(loc)
        totals.append(loc[:, _CW - 1:_CW])
    tot = jnp.concatenate(totals, axis=1)
    csum = tot
    shift = 1
    while shift < ncw:
        csum = csum + jnp.pad(csum, ((0, 0), (shift, 0)))[:, :ncw]
        shift *= 2
    carry = csum - tot

    v0 = xj0 - xi0
    v1 = xj1 - xi1
    v2 = xj2 - xi2
    dot = n0 * v0 + n1 * v1 + n2 * v2
    vsq = v0 * v0 + v1 * v1 + v2 * v2
    sqn = n0 * n0 + n1 * n1 + n2 * n2
    csq = sqn * vsq - dot * dot
    tiny = csq < 1e-20
    degen = tiny & (jnp.abs(dot) < 1e-10)
    denom = dot * dot + jnp.where(tiny, 0.0, csq)
    key = dot * jax.lax.rsqrt(jnp.where(degen, 1.0, denom))
    key = jnp.where(degen, 1.0, key)
    k_last = key[:, _N - 1]
    m = None
    for c in range(ncw):
        sl = slice(c * _CW, (c + 1) * _CW)
        inc_c = (locals_[c] + carry[:, c:c + 1] <= float(_K)) & mask[:, sl]
        mc_ = jnp.min(jnp.where(inc_c, key[:, sl], 2.0), axis=1)
        m = mc_ if m is None else jnp.minimum(m, mc_)
    m = jnp.clip(jnp.where(m > 1.5, k_last, m), -1.0, 1.0)
    sin = jnp.sqrt(jnp.maximum(1.0 - m * m, 0.0))
    out_ref[0, 0, :] = jnp.arctan2(sin, m)


@functools.partial(jax.jit, static_argnames=("interpret",))
def kernel(coords, normals, interpret=False):
    xiT = jnp.transpose(coords, (0, 2, 1))
    niT = jnp.transpose(normals, (0, 2, 1))
    grid = (_B, _N // _TI)
    out = pl.pallas_call(
        _ppf_tile,
        grid=grid,
        in_specs=[
            pl.BlockSpec((1, 3, _N), lambda b, t: (b, 0, 0)),
            pl.BlockSpec((1, _TI, 3), lambda b, t: (b, t, 0)),
            pl.BlockSpec((1, _TI, 3), lambda b, t: (b, t, 0)),
        ],
        out_specs=pl.BlockSpec((1, 1, _TI), lambda b, t: (b, 0, t)),
        out_shape=jax.ShapeDtypeStruct((_B, 1, _N), jnp.float32),
        compiler_params=pltpu.CompilerParams(
            dimension_semantics=("parallel", "parallel")),
        interpret=interpret,
    )(coords, xiT, niT)
    return out


# speedup vs baseline: 1.0420x; 1.0420x over previous
import functools

import jax
import jax.numpy as jnp
from jax.experimental import pallas as pl
from jax.experimental.pallas import tpu as pltpu

_B = 2
_N = 4096
_K = 32
_RADIUS = 0.15
_TI = 128
_CW = 512


def _ppf_tile(x_all_ref, xi_ref, ni_ref, out_ref):
    x_all = x_all_ref[0]
    xi = xi_ref[0]
    ni = ni_ref[0]

    xj0 = x_all[0:1, :]
    xj1 = x_all[1:2, :]
    xj2 = x_all[2:3, :]
    xi0 = xi[:, 0:1]
    xi1 = xi[:, 1:2]
    xi2 = xi[:, 2:3]
    n0 = ni[:, 0:1]
    n1 = ni[:, 1:2]
    n2 = ni[:, 2:3]

    sq_j = xj0 * xj0 + xj1 * xj1 + xj2 * xj2
    sq_i = xi0 * xi0 + xi1 * xi1 + xi2 * xi2
    dotx = jax.lax.dot_general(xi, x_all, (((1,), (0,)), ((), ())))
    d2 = jnp.maximum(sq_i + sq_j - 2.0 * dotx, 0.0)
    mask = d2 <= _RADIUS * _RADIUS

    ncw = _N // _CW
    tri = (jax.lax.broadcasted_iota(jnp.int32, (_CW, _CW), 0)
           <= jax.lax.broadcasted_iota(jnp.int32, (_CW, _CW), 1))
    tri = tri.astype(jnp.float32)
    locals_ = []
    totals = []
    for c in range(ncw):
        mc = mask[:, c * _CW:(c + 1) * _CW].astype(jnp.float32)
        loc = jax.lax.dot_general(mc, tri, (((1,), (0,)), ((), ())),
                                  preferred_element_type=jnp.float32)
        locals_.append(loc)
        totals.append(loc[:, _CW - 1:_CW])
    tot = jnp.concatenate(totals, axis=1)
    csum = tot
    shift = 1
    while shift < ncw:
        csum = csum + jnp.pad(csum, ((0, 0), (shift, 0)))[:, :ncw]
        shift *= 2
    carry = csum - tot

    v0 = xj0 - xi0
    v1 = xj1 - xi1
    v2 = xj2 - xi2
    dot = n0 * v0 + n1 * v1 + n2 * v2
    vsq = v0 * v0 + v1 * v1 + v2 * v2
    sqn = n0 * n0 + n1 * n1 + n2 * n2
    csq = sqn * vsq - dot * dot
    tiny = csq < 1e-20
    degen = tiny & (jnp.abs(dot) < 1e-10)
    denom = dot * dot + jnp.where(tiny, 0.0, csq)
    key = dot * jax.lax.rsqrt(jnp.where(degen, 1.0, denom))
    key = jnp.where(degen, 1.0, key)
    k_last = key[:, _N - 1]
    m = None
    for c in range(ncw):
        sl = slice(c * _CW, (c + 1) * _CW)
        inc_c = (locals_[c] + carry[:, c:c + 1] <= float(_K)) & mask[:, sl]
        mc_ = jnp.min(jnp.where(inc_c, key[:, sl], 2.0), axis=1)
        m = mc_ if m is None else jnp.minimum(m, mc_)
    m = jnp.clip(jnp.where(m > 1.5, k_last, m), -1.0, 1.0)
    sin = jnp.sqrt(jnp.maximum(1.0 - m * m, 0.0))
    out_ref[0, 0, :] = jnp.arctan2(sin, m)


@functools.partial(jax.jit, static_argnames=("interpret",))
def kernel(coords, normals, interpret=False):
    xiT = jnp.transpose(coords, (0, 2, 1))
    niT = jnp.transpose(normals, (0, 2, 1))
    grid = (_B, _N // _TI)
    out = pl.pallas_call(
        _ppf_tile,
        grid=grid,
        in_specs=[
            pl.BlockSpec((1, 3, _N), lambda b, t: (b, 0, 0)),
            pl.BlockSpec((1, _TI, 3), lambda b, t: (b, t, 0)),
            pl.BlockSpec((1, _TI, 3), lambda b, t: (b, t, 0)),
        ],
        out_specs=pl.BlockSpec((1, 1, _TI), lambda b, t: (b, 0, t)),
        out_shape=jax.ShapeDtypeStruct((_B, 1, _N), jnp.float32),
        compiler_params=pltpu.CompilerParams(
            dimension_semantics=("parallel", "parallel")),
        interpret=interpret,
    )(coords, xiT, niT)
    return out


# speedup vs baseline: 1.0565x; 1.0139x over previous
import functools

import jax
import jax.numpy as jnp
from jax.experimental import pallas as pl
from jax.experimental.pallas import tpu as pltpu

_B = 2
_N = 4096
_K = 32
_RADIUS = 0.15
_TI = 128
_CW = 512


def _ppf_tile(x_all_ref, xi_ref, ni_ref, out_ref):
    x_all = x_all_ref[0]
    xi = xi_ref[0]
    ni = ni_ref[0]

    xj0 = x_all[0:1, :]
    xj1 = x_all[1:2, :]
    xj2 = x_all[2:3, :]
    xi0 = xi[:, 0:1]
    xi1 = xi[:, 1:2]
    xi2 = xi[:, 2:3]
    n0 = ni[:, 0:1]
    n1 = ni[:, 1:2]
    n2 = ni[:, 2:3]

    sq_j = xj0 * xj0 + xj1 * xj1 + xj2 * xj2
    sq_i = xi0 * xi0 + xi1 * xi1 + xi2 * xi2
    dotx2 = jax.lax.dot_general(xi + xi, x_all, (((1,), (0,)), ((), ())))
    d2 = jnp.maximum(sq_i + sq_j - dotx2, 0.0)
    mask = d2 <= _RADIUS * _RADIUS

    ncw = _N // _CW
    tri = (jax.lax.broadcasted_iota(jnp.int32, (_CW, _CW), 0)
           <= jax.lax.broadcasted_iota(jnp.int32, (_CW, _CW), 1))
    tri = tri.astype(jnp.float32)
    locals_ = []
    totals = []
    for c in range(ncw):
        mc = mask[:, c * _CW:(c + 1) * _CW].astype(jnp.float32)
        loc = jax.lax.dot_general(mc, tri, (((1,), (0,)), ((), ())),
                                  preferred_element_type=jnp.float32)
        locals_.append(loc)
        totals.append(loc[:, _CW - 1:_CW])
    tot = jnp.concatenate(totals, axis=1)
    csum = tot
    shift = 1
    while shift < ncw:
        csum = csum + jnp.pad(csum, ((0, 0), (shift, 0)))[:, :ncw]
        shift *= 2
    carry = csum - tot

    v0 = xj0 - xi0
    v1 = xj1 - xi1
    v2 = xj2 - xi2
    dot = n0 * v0 + n1 * v1 + n2 * v2
    vsq = v0 * v0 + v1 * v1 + v2 * v2
    sqn = n0 * n0 + n1 * n1 + n2 * n2
    csq = sqn * vsq - dot * dot
    tiny = csq < 1e-20
    degen = tiny & (jnp.abs(dot) < 1e-10)
    denom = dot * dot + jnp.where(tiny, 0.0, csq)
    key = dot * jax.lax.rsqrt(jnp.where(degen, 1.0, denom))
    key = jnp.where(degen, 1.0, key)
    k_last = key[:, _N - 1]
    m = None
    for c in range(ncw):
        sl = slice(c * _CW, (c + 1) * _CW)
        inc_c = (locals_[c] + carry[:, c:c + 1] <= float(_K)) & mask[:, sl]
        mc_ = jnp.min(jnp.where(inc_c, key[:, sl], 2.0), axis=1)
        m = mc_ if m is None else jnp.minimum(m, mc_)
    m = jnp.clip(jnp.where(m > 1.5, k_last, m), -1.0, 1.0)
    sin = jnp.sqrt(jnp.maximum(1.0 - m * m, 0.0))
    out_ref[0, 0, :] = jnp.arctan2(sin, m)


@functools.partial(jax.jit, static_argnames=("interpret",))
def kernel(coords, normals, interpret=False):
    xiT = jnp.transpose(coords, (0, 2, 1))
    niT = jnp.transpose(normals, (0, 2, 1))
    grid = (_B, _N // _TI)
    out = pl.pallas_call(
        _ppf_tile,
        grid=grid,
        in_specs=[
            pl.BlockSpec((1, 3, _N), lambda b, t: (b, 0, 0)),
            pl.BlockSpec((1, _TI, 3), lambda b, t: (b, t, 0)),
            pl.BlockSpec((1, _TI, 3), lambda b, t: (b, t, 0)),
        ],
        out_specs=pl.BlockSpec((1, 1, _TI), lambda b, t: (b, 0, t)),
        out_shape=jax.ShapeDtypeStruct((_B, 1, _N), jnp.float32),
        compiler_params=pltpu.CompilerParams(
            dimension_semantics=("parallel", "parallel")),
        interpret=interpret,
    )(coords, xiT, niT)
    return out


# speedup vs baseline: 1.0752x; 1.0177x over previous
import functools

import jax
import jax.numpy as jnp
from jax.experimental import pallas as pl
from jax.experimental.pallas import tpu as pltpu

_B = 2
_N = 4096
_K = 32
_RADIUS = 0.15
_TI = 128
_CW = 512


def _ppf_tile(x_all_ref, xi_ref, ni_ref, out_ref):
    x_all = x_all_ref[0]
    xi = xi_ref[0]
    ni = ni_ref[0]

    xj0 = x_all[0:1, :]
    xj1 = x_all[1:2, :]
    xj2 = x_all[2:3, :]
    xi0 = xi[:, 0:1]
    xi1 = xi[:, 1:2]
    xi2 = xi[:, 2:3]
    n0 = ni[:, 0:1]
    n1 = ni[:, 1:2]
    n2 = ni[:, 2:3]

    sq_j = xj0 * xj0 + xj1 * xj1 + xj2 * xj2
    sq_i = xi0 * xi0 + xi1 * xi1 + xi2 * xi2
    dotx2 = jax.lax.dot_general(xi + xi, x_all, (((1,), (0,)), ((), ())))
    d2 = jnp.maximum(sq_i + sq_j - dotx2, 0.0)
    mask = d2 <= _RADIUS * _RADIUS

    ncw = _N // _CW
    tri = (jax.lax.broadcasted_iota(jnp.int32, (_CW, _CW), 0)
           <= jax.lax.broadcasted_iota(jnp.int32, (_CW, _CW), 1))
    tri = tri.astype(jnp.float32)
    locals_ = []
    totals = []
    for c in range(ncw):
        mc = mask[:, c * _CW:(c + 1) * _CW].astype(jnp.float32)
        loc = jax.lax.dot_general(mc, tri, (((1,), (0,)), ((), ())),
                                  preferred_element_type=jnp.float32)
        locals_.append(loc)
        totals.append(loc[:, _CW - 1:_CW])
    tot = jnp.concatenate(totals, axis=1)
    csum = tot
    shift = 1
    while shift < ncw:
        csum = csum + jnp.pad(csum, ((0, 0), (shift, 0)))[:, :ncw]
        shift *= 2
    carry = csum - tot

    v0 = xj0 - xi0
    v1 = xj1 - xi1
    v2 = xj2 - xi2
    dot = n0 * v0 + n1 * v1 + n2 * v2
    vsq = v0 * v0 + v1 * v1 + v2 * v2
    sqn = n0 * n0 + n1 * n1 + n2 * n2
    csq = sqn * vsq - dot * dot
    tiny = csq < 1e-20
    degen = tiny & (jnp.abs(dot) < 1e-10)
    denom = dot * dot + jnp.where(tiny, 0.0, csq)
    key = dot * jax.lax.rsqrt(denom)
    key = jnp.where(degen, 1.0, key)
    k_last = key[:, _N - 1]
    m = None
    for c in range(ncw):
        sl = slice(c * _CW, (c + 1) * _CW)
        inc_c = (locals_[c] + carry[:, c:c + 1] <= float(_K)) & mask[:, sl]
        mc_ = jnp.min(jnp.where(inc_c, key[:, sl], 2.0), axis=1)
        m = mc_ if m is None else jnp.minimum(m, mc_)
    m = jnp.clip(jnp.where(m > 1.5, k_last, m), -1.0, 1.0)
    sin = jnp.sqrt(jnp.maximum(1.0 - m * m, 0.0))
    out_ref[0, 0, :] = jnp.arctan2(sin, m)


@functools.partial(jax.jit, static_argnames=("interpret",))
def kernel(coords, normals, interpret=False):
    xiT = jnp.transpose(coords, (0, 2, 1))
    niT = jnp.transpose(normals, (0, 2, 1))
    grid = (_B, _N // _TI)
    out = pl.pallas_call(
        _ppf_tile,
        grid=grid,
        in_specs=[
            pl.BlockSpec((1, 3, _N), lambda b, t: (b, 0, 0)),
            pl.BlockSpec((1, _TI, 3), lambda b, t: (b, t, 0)),
            pl.BlockSpec((1, _TI, 3), lambda b, t: (b, t, 0)),
        ],
        out_specs=pl.BlockSpec((1, 1, _TI), lambda b, t: (b, 0, t)),
        out_shape=jax.ShapeDtypeStruct((_B, 1, _N), jnp.float32),
        compiler_params=pltpu.CompilerParams(
            dimension_semantics=("parallel", "parallel")),
        interpret=interpret,
    )(coords, xiT, niT)
    return out


# speedup vs baseline: 1.0894x; 1.0132x over previous
import functools

import jax
import jax.numpy as jnp
from jax.experimental import pallas as pl
from jax.experimental.pallas import tpu as pltpu

_B = 2
_N = 4096
_K = 32
_RADIUS = 0.15
_TI = 128
_CW = 512


def _ppf_tile(x_all_ref, xi_ref, ni_ref, out_ref):
    x_all = x_all_ref[0]
    xi = xi_ref[0]
    ni = ni_ref[0]

    xj0 = x_all[0:1, :]
    xj1 = x_all[1:2, :]
    xj2 = x_all[2:3, :]
    xi0 = xi[:, 0:1]
    xi1 = xi[:, 1:2]
    xi2 = xi[:, 2:3]
    n0 = ni[:, 0:1]
    n1 = ni[:, 1:2]
    n2 = ni[:, 2:3]

    sq_j = xj0 * xj0 + xj1 * xj1 + xj2 * xj2
    sq_i = xi0 * xi0 + xi1 * xi1 + xi2 * xi2
    dotx2 = jax.lax.dot_general(xi + xi, x_all, (((1,), (0,)), ((), ())))
    d2 = jnp.maximum(sq_i + sq_j - dotx2, 0.0)
    mask = d2 <= _RADIUS * _RADIUS

    ncw = _N // _CW
    tri = (jax.lax.broadcasted_iota(jnp.int32, (_CW, _CW), 0)
           <= jax.lax.broadcasted_iota(jnp.int32, (_CW, _CW), 1))
    tri = tri.astype(jnp.float32)
    locals_ = []
    totals = []
    for c in range(ncw):
        mc = mask[:, c * _CW:(c + 1) * _CW].astype(jnp.float32)
        loc = jax.lax.dot_general(mc, tri, (((1,), (0,)), ((), ())),
                                  preferred_element_type=jnp.float32)
        locals_.append(loc)
        totals.append(loc[:, _CW - 1:_CW])
    tot = jnp.concatenate(totals, axis=1)
    csum = tot
    shift = 1
    while shift < ncw:
        csum = csum + jnp.pad(csum, ((0, 0), (shift, 0)))[:, :ncw]
        shift *= 2
    carry = csum - tot

    v0 = xj0 - xi0
    v1 = xj1 - xi1
    v2 = xj2 - xi2
    dot = n0 * v0 + n1 * v1 + n2 * v2
    vsq = v0 * v0 + v1 * v1 + v2 * v2
    sqn = n0 * n0 + n1 * n1 + n2 * n2
    csq = sqn * vsq - dot * dot
    tiny = csq < 1e-20
    degen = tiny & (dot * dot < 1e-20)
    denom = dot * dot + jnp.where(tiny, 0.0, csq)
    key = dot * jax.lax.rsqrt(denom)
    key = jnp.where(degen, 1.0, key)
    k_last = key[:, _N - 1]
    m = None
    for c in range(ncw):
        sl = slice(c * _CW, (c + 1) * _CW)
        inc_c = (locals_[c] + carry[:, c:c + 1] <= float(_K)) & mask[:, sl]
        mc_ = jnp.min(jnp.where(inc_c, key[:, sl], 2.0), axis=1)
        m = mc_ if m is None else jnp.minimum(m, mc_)
    m = jnp.clip(jnp.where(m > 1.5, k_last, m), -1.0, 1.0)
    sin = jnp.sqrt(jnp.maximum(1.0 - m * m, 0.0))
    out_ref[0, 0, :] = jnp.arctan2(sin, m)


@functools.partial(jax.jit, static_argnames=("interpret",))
def kernel(coords, normals, interpret=False):
    xiT = jnp.transpose(coords, (0, 2, 1))
    niT = jnp.transpose(normals, (0, 2, 1))
    grid = (_B, _N // _TI)
    out = pl.pallas_call(
        _ppf_tile,
        grid=grid,
        in_specs=[
            pl.BlockSpec((1, 3, _N), lambda b, t: (b, 0, 0)),
            pl.BlockSpec((1, _TI, 3), lambda b, t: (b, t, 0)),
            pl.BlockSpec((1, _TI, 3), lambda b, t: (b, t, 0)),
        ],
        out_specs=pl.BlockSpec((1, 1, _TI), lambda b, t: (b, 0, t)),
        out_shape=jax.ShapeDtypeStruct((_B, 1, _N), jnp.float32),
        compiler_params=pltpu.CompilerParams(
            dimension_semantics=("parallel", "parallel")),
        interpret=interpret,
    )(coords, xiT, niT)
    return out
